# baseline (speedup 1.0000x reference)
import jax
import jax.numpy as jnp
from jax.experimental import pallas as pl
from jax.experimental.pallas import tpu as pltpu

_EPS = 1e-5


def _ln(x, g, b):
    m = jnp.mean(x, axis=-1, keepdims=True)
    v = jnp.mean((x - m) ** 2, axis=-1, keepdims=True)
    return (x - m) * jax.lax.rsqrt(v + _EPS) * g + b


def _mlp(x, p):
    ws, bs, lng, lnb = p["ws"], p["bs"], p["lng"], p["lnb"]
    n = len(ws)
    for i in range(n):
        w = ws[i]
        if w.shape[0] == 1:
            x = x * w + bs[i]
        else:
            x = x @ w + bs[i]
        if i < n - 1:
            x = _ln(jnp.maximum(x, 0.0), lng[i], lnb[i])
    return x


def _prep_params(params):
    def row(b):
        return b.reshape(1, -1)

    pp = {}
    for name in ("node_init", "dis_init", "node_mlp"):
        p = params[name]
        pp[name] = {
            "ws": list(p["ws"]),
            "bs": [row(b) for b in p["bs"]],
            "lng": [row(g) for g in p["lng"]],
            "lnb": [row(b) for b in p["lnb"]],
        }
    for name in ("query", "key", "value"):
        pp[name] = {"w": params[name]["w"], "b": row(params[name]["b"])}
    em = params["edge_mlp"]
    w1 = em["ws"][0]
    w2 = em["ws"][1][:, 0]
    pp["edge"] = {
        "w1a": w1[0:32],
        "w1b": w1[32:64],
        "w1c": w1[64:96],
        "b1": row(em["bs"][0]),
        "gw2": (em["lng"][0] * w2).reshape(1, 32, 1),
    }
    return pp


def _forward(pp, ghost, agent, dis1, lghost, lagent, out_ref, oi):
    N, A, G = ghost.shape[0], agent.shape[0], lghost.shape[0]
    C = 32

    e = pp["edge"]
    w1a, w1b, w1c = e["w1a"], e["w1b"], e["w1c"]
    b1, gw2 = e["b1"], e["gw2"]
    g1 = jnp.sum(gw2)
    csum = jnp.sum(w1c, axis=0, keepdims=True)

    def lin(x, p):
        return x @ p["w"] + p["b"]

    def edge_scores(u, vT, dT, chunk):
        nx = u.shape[0]
        outs = []
        for s in range(0, nx, chunk):
            nc = min(chunk, nx - s)
            h = u[s:s + nc][:, :, None] + vT[None, :, :]
            if dT is not None:
                h = h + dT[s:s + nc]
            r = jnp.maximum(h, 0.0)
            s1 = jnp.mean(r, axis=1)
            s2 = jnp.mean(r * r, axis=1)
            sw = jnp.sum(r * gw2, axis=1)
            var = s2 - s1 * s1
            outs.append((sw - s1 * g1) * jax.lax.rsqrt(var + _EPS))
        return outs[0] if len(outs) == 1 else jnp.concatenate(outs, axis=0)

    def intra(x):
        q, k, v = lin(x, pp["query"]), lin(x, pp["key"]), lin(x, pp["value"])
        s = jax.lax.dot_general(q, k, (((1,), (1,)), ((), ())))
        att = jax.nn.softmax(s, axis=-1)
        h = jnp.concatenate([x, att @ v], axis=-1)
        return x + _mlp(h, pp["node_mlp"])

    def inter(x, y, dT, need_x, need_y, need_e, chunk):
        base = b1 if dT is not None else b1 + csum
        e1 = x_new = y_new = None
        if need_x or need_e:
            u = lin(x, pp["query"]) @ w1a + base
            vT = (lin(y, pp["key"]) @ w1b).T
            e1 = jax.nn.softmax(edge_scores(u, vT, dT, chunk), axis=-1)
        if need_x:
            yv = lin(y, pp["value"])
            x_new = x + _mlp(jnp.concatenate([x, e1 @ yv], axis=-1),
                             pp["node_mlp"])
        if need_y:
            u2 = lin(x, pp["key"]) @ w1b + base
            vT2 = (lin(y, pp["query"]) @ w1a).T
            s2 = edge_scores(u2, vT2, dT, chunk).T
            e2 = jax.nn.softmax(s2, axis=-1)
            xv = lin(x, pp["value"])
            y_new = y + _mlp(jnp.concatenate([y, e2 @ xv], axis=-1),
                             pp["node_mlp"])
        return e1, x_new, y_new

    x0 = jnp.concatenate([ghost, lghost, lagent, agent], axis=0)
    h0 = _mlp(x0, pp["node_init"])
    fx, lfx, lax_, ax = (h0[0:N], h0[N:N + G], h0[N + G:N + 2 * G],
                         h0[N + 2 * G:N + 2 * G + A])
    dis = _mlp(dis1, pp["dis_init"])
    dT = (dis @ w1c).reshape(A, N, C).transpose(0, 2, 1)

    e_all = None
    for it in range(3):
        last = it == 2
        fx, ax = intra(fx), intra(ax)
        lfx, lax_ = intra(lfx), intra(lax_)
        _, nlfx, fx = inter(lfx, fx, None, not last, True, False, 128)
        _, nlax, ax = inter(lax_, ax, None, not last, True, False, 128)
        if not last:
            lfx, lax_ = nlfx, nlax
        e_all, nax, nfx = inter(ax, fx, dT, not last, not last, True, 128)
        if not last:
            ax, fx = nax, nfx
    out_ref[oi] = e_all


def kernel(ghost_node_position, agent_world_pos, graph_agent_dis,
           last_ghost_node_position, last_agent_world_pos, params):
    B, N, _ = ghost_node_position.shape
    A = agent_world_pos.shape[1]
    G = last_ghost_node_position.shape[1]

    pp = _prep_params(params)
    wleaves, treedef = jax.tree_util.tree_flatten(pp)
    dis1 = graph_agent_dis.reshape(B, A * N, 1)

    PB = 4 if B % 4 == 0 else (2 if B % 2 == 0 else 1)

    def body(gref, aref, dref, lgref, laref, *rest):
        wrefs, out_ref = rest[:-1], rest[-1]
        p = jax.tree_util.tree_unflatten(treedef, [r[...] for r in wrefs])
        for i in range(PB):
            _forward(p, gref[i], aref[i], dref[i], lgref[i], laref[i],
                     out_ref, i)

    def bspec(shape):
        nd = len(shape)
        return pl.BlockSpec((PB,) + shape[1:],
                            lambda b, _n=nd: (b,) + (0,) * (_n - 1))

    def wspec(w):
        nd = w.ndim
        return pl.BlockSpec(w.shape, lambda b, _n=nd: (0,) * _n)

    in_specs = [
        bspec((B, N, 4)), bspec((B, A, 4)), bspec((B, A * N, 1)),
        bspec((B, G, 4)), bspec((B, G, 4)),
    ] + [wspec(w) for w in wleaves]

    return pl.pallas_call(
        body,
        grid=(B // PB,),
        in_specs=in_specs,
        out_specs=pl.BlockSpec((PB, A, N), lambda b: (b, 0, 0)),
        out_shape=jax.ShapeDtypeStruct((B, A, N), jnp.float32),
        compiler_params=pltpu.CompilerParams(
            dimension_semantics=("parallel",)),
    )(ghost_node_position, agent_world_pos, dis1,
      last_ghost_node_position, last_agent_world_pos, *wleaves)


# speedup vs baseline: 1.0148x; 1.0148x over previous
import jax
import jax.numpy as jnp
from jax.experimental import pallas as pl
from jax.experimental.pallas import tpu as pltpu

_EPS = 1e-5


def _ln(x, g, b):
    m = jnp.mean(x, axis=-1, keepdims=True)
    v = jnp.mean((x - m) ** 2, axis=-1, keepdims=True)
    return (x - m) * jax.lax.rsqrt(v + _EPS) * g + b


def _mlp(x, p):
    ws, bs, lng, lnb = p["ws"], p["bs"], p["lng"], p["lnb"]
    n = len(ws)
    for i in range(n):
        w = ws[i]
        if w.shape[0] == 1:
            x = x * w + bs[i]
        else:
            x = x @ w + bs[i]
        if i < n - 1:
            x = _ln(jnp.maximum(x, 0.0), lng[i], lnb[i])
    return x


def _prep_params(params):
    def row(b):
        return b.reshape(1, -1)

    pp = {}
    for name in ("node_init", "dis_init", "node_mlp"):
        p = params[name]
        pp[name] = {
            "ws": list(p["ws"]),
            "bs": [row(b) for b in p["bs"]],
            "lng": [row(g) for g in p["lng"]],
            "lnb": [row(b) for b in p["lnb"]],
        }
    for name in ("query", "key", "value"):
        pp[name] = {"w": params[name]["w"], "b": row(params[name]["b"])}
    em = params["edge_mlp"]
    w1 = em["ws"][0]
    w2 = em["ws"][1][:, 0]
    pp["edge"] = {
        "w1a": w1[0:32],
        "w1b": w1[32:64],
        "w1c": w1[64:96],
        "b1": row(em["bs"][0]),
        "gw2": (em["lng"][0] * w2).reshape(1, 32, 1),
    }
    return pp


def _forward(pp, ghost, agent, dis1, lghost, lagent, out_ref, oi):
    N, A, G = ghost.shape[0], agent.shape[0], lghost.shape[0]
    C = 32

    e = pp["edge"]
    w1a, w1b, w1c = e["w1a"], e["w1b"], e["w1c"]
    b1, gw2 = e["b1"], e["gw2"]
    g1 = jnp.sum(gw2)
    csum = jnp.sum(w1c, axis=0, keepdims=True)

    def lin(x, p):
        return x @ p["w"] + p["b"]

    def edge_scores(u, vT, dT, chunk):
        nx = u.shape[0]
        outs = []
        for s in range(0, nx, chunk):
            nc = min(chunk, nx - s)
            h = u[s:s + nc][:, :, None] + vT[None, :, :]
            if dT is not None:
                h = h + dT[s:s + nc]
            r = jnp.maximum(h, 0.0)
            s1 = jnp.mean(r, axis=1)
            s2 = jnp.mean(r * r, axis=1)
            sw = jnp.sum(r * gw2, axis=1)
            var = s2 - s1 * s1
            outs.append((sw - s1 * g1) * jax.lax.rsqrt(var + _EPS))
        return outs[0] if len(outs) == 1 else jnp.concatenate(outs, axis=0)

    def intra(x):
        q, k, v = lin(x, pp["query"]), lin(x, pp["key"]), lin(x, pp["value"])
        s = jax.lax.dot_general(q, k, (((1,), (1,)), ((), ())))
        att = jax.nn.softmax(s, axis=-1)
        h = jnp.concatenate([x, att @ v], axis=-1)
        return x + _mlp(h, pp["node_mlp"])

    def inter(x, y, dT, need_x, need_y, need_e, chunk):
        base = b1 if dT is not None else b1 + csum
        e1 = x_new = y_new = None
        if need_x or need_e:
            u = lin(x, pp["query"]) @ w1a + base
            vT = (lin(y, pp["key"]) @ w1b).T
            e1 = jax.nn.softmax(edge_scores(u, vT, dT, chunk), axis=-1)
        if need_x:
            yv = lin(y, pp["value"])
            x_new = x + _mlp(jnp.concatenate([x, e1 @ yv], axis=-1),
                             pp["node_mlp"])
        if need_y:
            u2 = lin(x, pp["key"]) @ w1b + base
            vT2 = (lin(y, pp["query"]) @ w1a).T
            s2 = edge_scores(u2, vT2, dT, chunk).T
            e2 = jax.nn.softmax(s2, axis=-1)
            xv = lin(x, pp["value"])
            y_new = y + _mlp(jnp.concatenate([y, e2 @ xv], axis=-1),
                             pp["node_mlp"])
        return e1, x_new, y_new

    x0 = jnp.concatenate([ghost, lghost, lagent, agent], axis=0)
    h0 = _mlp(x0, pp["node_init"])
    fx, lfx, lax_, ax = (h0[0:N], h0[N:N + G], h0[N + G:N + 2 * G],
                         h0[N + 2 * G:N + 2 * G + A])
    dis = _mlp(dis1, pp["dis_init"])
    dT = (dis @ w1c).reshape(A, N, C).transpose(0, 2, 1)

    e_all = None
    for it in range(3):
        last = it == 2
        fx, ax = intra(fx), intra(ax)
        lfx, lax_ = intra(lfx), intra(lax_)
        _, nlfx, fx = inter(lfx, fx, None, not last, True, False, 32)
        _, nlax, ax = inter(lax_, ax, None, not last, True, False, 128)
        if not last:
            lfx, lax_ = nlfx, nlax
        e_all, nax, nfx = inter(ax, fx, dT, not last, not last, True, 128)
        if not last:
            ax, fx = nax, nfx
    out_ref[oi] = e_all


def kernel(ghost_node_position, agent_world_pos, graph_agent_dis,
           last_ghost_node_position, last_agent_world_pos, params):
    B, N, _ = ghost_node_position.shape
    A = agent_world_pos.shape[1]
    G = last_ghost_node_position.shape[1]

    pp = _prep_params(params)
    wleaves, treedef = jax.tree_util.tree_flatten(pp)
    dis1 = graph_agent_dis.reshape(B, A * N, 1)

    PB = 4 if B % 4 == 0 else (2 if B % 2 == 0 else 1)

    def body(gref, aref, dref, lgref, laref, *rest):
        wrefs, out_ref = rest[:-1], rest[-1]
        p = jax.tree_util.tree_unflatten(treedef, [r[...] for r in wrefs])
        for i in range(PB):
            _forward(p, gref[i], aref[i], dref[i], lgref[i], laref[i],
                     out_ref, i)

    def bspec(shape):
        nd = len(shape)
        return pl.BlockSpec((PB,) + shape[1:],
                            lambda b, _n=nd: (b,) + (0,) * (_n - 1))

    def wspec(w):
        nd = w.ndim
        return pl.BlockSpec(w.shape, lambda b, _n=nd: (0,) * _n)

    in_specs = [
        bspec((B, N, 4)), bspec((B, A, 4)), bspec((B, A * N, 1)),
        bspec((B, G, 4)), bspec((B, G, 4)),
    ] + [wspec(w) for w in wleaves]

    return pl.pallas_call(
        body,
        grid=(B // PB,),
        in_specs=in_specs,
        out_specs=pl.BlockSpec((PB, A, N), lambda b: (b, 0, 0)),
        out_shape=jax.ShapeDtypeStruct((B, A, N), jnp.float32),
        compiler_params=pltpu.CompilerParams(
            dimension_semantics=("parallel",)),
    )(ghost_node_position, agent_world_pos, dis1,
      last_ghost_node_position, last_agent_world_pos, *wleaves)


# speedup vs baseline: 1.0152x; 1.0004x over previous
import jax
import jax.numpy as jnp
from jax.experimental import pallas as pl
from jax.experimental.pallas import tpu as pltpu

_EPS = 1e-5


def _ln(x, g, b):
    m = jnp.mean(x, axis=-1, keepdims=True)
    v = jnp.mean((x - m) ** 2, axis=-1, keepdims=True)
    return (x - m) * jax.lax.rsqrt(v + _EPS) * g + b


def _mlp(x, p):
    ws, bs, lng, lnb = p["ws"], p["bs"], p["lng"], p["lnb"]
    n = len(ws)
    for i in range(n):
        w = ws[i]
        if w.shape[0] == 1:
            x = x * w + bs[i]
        else:
            x = x @ w + bs[i]
        if i < n - 1:
            x = _ln(jnp.maximum(x, 0.0), lng[i], lnb[i])
    return x


def _prep_params(params):
    def row(b):
        return b.reshape(1, -1)

    pp = {}
    for name in ("node_init", "dis_init", "node_mlp"):
        p = params[name]
        pp[name] = {
            "ws": list(p["ws"]),
            "bs": [row(b) for b in p["bs"]],
            "lng": [row(g) for g in p["lng"]],
            "lnb": [row(b) for b in p["lnb"]],
        }
    for name in ("query", "key", "value"):
        pp[name] = {"w": params[name]["w"], "b": row(params[name]["b"])}
    em = params["edge_mlp"]
    w1 = em["ws"][0]
    w2 = em["ws"][1][:, 0]
    pp["edge"] = {
        "w1a": w1[0:32],
        "w1b": w1[32:64],
        "w1c": w1[64:96],
        "b1": row(em["bs"][0]),
        "gw2": (em["lng"][0] * w2).reshape(1, 32, 1),
    }
    return pp


def _forward(pp, ghost, agent, dis1, lghost, lagent, out_ref, oi):
    N, A, G = ghost.shape[0], agent.shape[0], lghost.shape[0]
    C = 32

    e = pp["edge"]
    w1a, w1b, w1c = e["w1a"], e["w1b"], e["w1c"]
    b1, gw2 = e["b1"], e["gw2"]
    g1 = jnp.sum(gw2)
    csum = jnp.sum(w1c, axis=0, keepdims=True)

    def lin(x, p):
        return x @ p["w"] + p["b"]

    def edge_scores(u, vT, dT, chunk):
        nx = u.shape[0]
        outs = []
        for s in range(0, nx, chunk):
            nc = min(chunk, nx - s)
            h = u[s:s + nc][:, :, None] + vT[None, :, :]
            if dT is not None:
                h = h + dT[s:s + nc]
            r = jnp.maximum(h, 0.0)
            s1 = jnp.mean(r, axis=1)
            s2 = jnp.mean(r * r, axis=1)
            sw = jnp.sum(r * gw2, axis=1)
            var = s2 - s1 * s1
            outs.append((sw - s1 * g1) * jax.lax.rsqrt(var + _EPS))
        return outs[0] if len(outs) == 1 else jnp.concatenate(outs, axis=0)

    def intra(x):
        q, k, v = lin(x, pp["query"]), lin(x, pp["key"]), lin(x, pp["value"])
        s = jax.lax.dot_general(q, k, (((1,), (1,)), ((), ())))
        att = jax.nn.softmax(s, axis=-1)
        h = jnp.concatenate([x, att @ v], axis=-1)
        return x + _mlp(h, pp["node_mlp"])

    def inter(x, y, dT, need_x, need_y, need_e, chunk):
        base = b1 if dT is not None else b1 + csum
        e1 = x_new = y_new = None
        if need_x or need_e:
            u = lin(x, pp["query"]) @ w1a + base
            vT = (lin(y, pp["key"]) @ w1b).T
            e1 = jax.nn.softmax(edge_scores(u, vT, dT, chunk), axis=-1)
        if need_x:
            yv = lin(y, pp["value"])
            x_new = x + _mlp(jnp.concatenate([x, e1 @ yv], axis=-1),
                             pp["node_mlp"])
        if need_y:
            u2 = lin(x, pp["key"]) @ w1b + base
            vT2 = (lin(y, pp["query"]) @ w1a).T
            s2 = edge_scores(u2, vT2, dT, chunk).T
            e2 = jax.nn.softmax(s2, axis=-1)
            xv = lin(x, pp["value"])
            y_new = y + _mlp(jnp.concatenate([y, e2 @ xv], axis=-1),
                             pp["node_mlp"])
        return e1, x_new, y_new

    x0 = jnp.concatenate([ghost, lghost, lagent, agent], axis=0)
    h0 = _mlp(x0, pp["node_init"])
    fx, lfx, lax_, ax = (h0[0:N], h0[N:N + G], h0[N + G:N + 2 * G],
                         h0[N + 2 * G:N + 2 * G + A])
    dis = _mlp(dis1, pp["dis_init"])
    dT = (dis @ w1c).reshape(A, N, C).transpose(0, 2, 1)

    e_all = None
    for it in range(3):
        last = it == 2
        fx, ax = intra(fx), intra(ax)
        lfx, lax_ = intra(lfx), intra(lax_)
        _, nlfx, fx = inter(lfx, fx, None, not last, True, False, 64)
        _, nlax, ax = inter(lax_, ax, None, not last, True, False, 128)
        if not last:
            lfx, lax_ = nlfx, nlax
        e_all, nax, nfx = inter(ax, fx, dT, not last, not last, True, 128)
        if not last:
            ax, fx = nax, nfx
    out_ref[oi] = e_all


def kernel(ghost_node_position, agent_world_pos, graph_agent_dis,
           last_ghost_node_position, last_agent_world_pos, params):
    B, N, _ = ghost_node_position.shape
    A = agent_world_pos.shape[1]
    G = last_ghost_node_position.shape[1]

    pp = _prep_params(params)
    wleaves, treedef = jax.tree_util.tree_flatten(pp)
    dis1 = graph_agent_dis.reshape(B, A * N, 1)

    PB = 4 if B % 4 == 0 else (2 if B % 2 == 0 else 1)

    def body(gref, aref, dref, lgref, laref, *rest):
        wrefs, out_ref = rest[:-1], rest[-1]
        p = jax.tree_util.tree_unflatten(treedef, [r[...] for r in wrefs])
        for i in range(PB):
            _forward(p, gref[i], aref[i], dref[i], lgref[i], laref[i],
                     out_ref, i)

    def bspec(shape):
        nd = len(shape)
        return pl.BlockSpec((PB,) + shape[1:],
                            lambda b, _n=nd: (b,) + (0,) * (_n - 1))

    def wspec(w):
        nd = w.ndim
        return pl.BlockSpec(w.shape, lambda b, _n=nd: (0,) * _n)

    in_specs = [
        bspec((B, N, 4)), bspec((B, A, 4)), bspec((B, A * N, 1)),
        bspec((B, G, 4)), bspec((B, G, 4)),
    ] + [wspec(w) for w in wleaves]

    return pl.pallas_call(
        body,
        grid=(B // PB,),
        in_specs=in_specs,
        out_specs=pl.BlockSpec((PB, A, N), lambda b: (b, 0, 0)),
        out_shape=jax.ShapeDtypeStruct((B, A, N), jnp.float32),
        compiler_params=pltpu.CompilerParams(
            dimension_semantics=("parallel",)),
    )(ghost_node_position, agent_world_pos, dis1,
      last_ghost_node_position, last_agent_world_pos, *wleaves)


# speedup vs baseline: 1.0230x; 1.0077x over previous
import jax
import jax.numpy as jnp
from jax.experimental import pallas as pl
from jax.experimental.pallas import tpu as pltpu

_EPS = 1e-5


def _ln(x, g, b):
    m = jnp.mean(x, axis=-1, keepdims=True)
    v = jnp.mean((x - m) ** 2, axis=-1, keepdims=True)
    return (x - m) * jax.lax.rsqrt(v + _EPS) * g + b


def _mlp(x, p):
    ws, bs, lng, lnb = p["ws"], p["bs"], p["lng"], p["lnb"]
    n = len(ws)
    for i in range(n):
        w = ws[i]
        if w.shape[0] == 1:
            x = x * w + bs[i]
        else:
            x = x @ w + bs[i]
        if i < n - 1:
            x = _ln(jnp.maximum(x, 0.0), lng[i], lnb[i])
    return x


def _prep_params(params):
    def row(b):
        return b.reshape(1, -1)

    pp = {}
    for name in ("node_init", "dis_init", "node_mlp"):
        p = params[name]
        pp[name] = {
            "ws": list(p["ws"]),
            "bs": [row(b) for b in p["bs"]],
            "lng": [row(g) for g in p["lng"]],
            "lnb": [row(b) for b in p["lnb"]],
        }
    for name in ("query", "key", "value"):
        pp[name] = {"w": params[name]["w"], "b": row(params[name]["b"])}
    em = params["edge_mlp"]
    w1 = em["ws"][0]
    w2 = em["ws"][1][:, 0]
    pp["edge"] = {
        "w1a": w1[0:32],
        "w1b": w1[32:64],
        "w1c": w1[64:96],
        "b1": row(em["bs"][0]),
        "gw2": (em["lng"][0] * w2).reshape(1, 32, 1),
    }
    return pp


def _forward(pp, ghost, agent, dis1, lghost, lagent, out_ref, oi):
    N, A, G = ghost.shape[0], agent.shape[0], lghost.shape[0]
    C = 32

    e = pp["edge"]
    w1a, w1b, w1c = e["w1a"], e["w1b"], e["w1c"]
    b1, gw2 = e["b1"], e["gw2"]
    g1 = jnp.sum(gw2)
    csum = jnp.sum(w1c, axis=0, keepdims=True)
    wqa = pp["query"]["w"] @ w1a
    bqa = pp["query"]["b"] @ w1a
    wkb = pp["key"]["w"] @ w1b
    bkb = pp["key"]["b"] @ w1b

    def lin(x, p):
        return x @ p["w"] + p["b"]

    def edge_scores(u, vT, dT, chunk):
        nx = u.shape[0]
        outs = []
        for s in range(0, nx, chunk):
            nc = min(chunk, nx - s)
            h = u[s:s + nc][:, :, None] + vT[None, :, :]
            if dT is not None:
                h = h + dT[s:s + nc]
            r = jnp.maximum(h, 0.0)
            s1 = jnp.mean(r, axis=1)
            s2 = jnp.mean(r * r, axis=1)
            sw = jnp.sum(r * gw2, axis=1)
            var = s2 - s1 * s1
            outs.append((sw - s1 * g1) * jax.lax.rsqrt(var + _EPS))
        return outs[0] if len(outs) == 1 else jnp.concatenate(outs, axis=0)

    def intra(x):
        q, k, v = lin(x, pp["query"]), lin(x, pp["key"]), lin(x, pp["value"])
        s = jax.lax.dot_general(q, k, (((1,), (1,)), ((), ())))
        att = jax.nn.softmax(s, axis=-1)
        h = jnp.concatenate([x, att @ v], axis=-1)
        return x + _mlp(h, pp["node_mlp"])

    def inter(x, y, dT, need_x, need_y, need_e, chunk):
        base = b1 if dT is not None else b1 + csum
        e1 = x_new = y_new = None
        if need_x or need_e:
            u = x @ wqa + (bqa + base)
            vT = (y @ wkb + bkb).T
            e1 = jax.nn.softmax(edge_scores(u, vT, dT, chunk), axis=-1)
        if need_x:
            yv = lin(y, pp["value"])
            x_new = x + _mlp(jnp.concatenate([x, e1 @ yv], axis=-1),
                             pp["node_mlp"])
        if need_y:
            u2 = x @ wkb + (bkb + base)
            vT2 = (y @ wqa + bqa).T
            s2 = edge_scores(u2, vT2, dT, chunk).T
            e2 = jax.nn.softmax(s2, axis=-1)
            xv = lin(x, pp["value"])
            y_new = y + _mlp(jnp.concatenate([y, e2 @ xv], axis=-1),
                             pp["node_mlp"])
        return e1, x_new, y_new

    x0 = jnp.concatenate([ghost, lghost, lagent, agent], axis=0)
    h0 = _mlp(x0, pp["node_init"])
    fx, lfx, lax_, ax = (h0[0:N], h0[N:N + G], h0[N + G:N + 2 * G],
                         h0[N + 2 * G:N + 2 * G + A])
    dis = _mlp(dis1, pp["dis_init"])
    dT = (dis @ w1c).reshape(A, N, C).transpose(0, 2, 1)

    e_all = None
    for it in range(3):
        last = it == 2
        fx, ax = intra(fx), intra(ax)
        lfx, lax_ = intra(lfx), intra(lax_)
        _, nlfx, fx = inter(lfx, fx, None, not last, True, False, 64)
        _, nlax, ax = inter(lax_, ax, None, not last, True, False, 128)
        if not last:
            lfx, lax_ = nlfx, nlax
        e_all, nax, nfx = inter(ax, fx, dT, not last, not last, True, 128)
        if not last:
            ax, fx = nax, nfx
    out_ref[oi] = e_all


def kernel(ghost_node_position, agent_world_pos, graph_agent_dis,
           last_ghost_node_position, last_agent_world_pos, params):
    B, N, _ = ghost_node_position.shape
    A = agent_world_pos.shape[1]
    G = last_ghost_node_position.shape[1]

    pp = _prep_params(params)
    wleaves, treedef = jax.tree_util.tree_flatten(pp)
    dis1 = graph_agent_dis.reshape(B, A * N, 1)

    PB = 4 if B % 4 == 0 else (2 if B % 2 == 0 else 1)

    def body(gref, aref, dref, lgref, laref, *rest):
        wrefs, out_ref = rest[:-1], rest[-1]
        p = jax.tree_util.tree_unflatten(treedef, [r[...] for r in wrefs])
        for i in range(PB):
            _forward(p, gref[i], aref[i], dref[i], lgref[i], laref[i],
                     out_ref, i)

    def bspec(shape):
        nd = len(shape)
        return pl.BlockSpec((PB,) + shape[1:],
                            lambda b, _n=nd: (b,) + (0,) * (_n - 1))

    def wspec(w):
        nd = w.ndim
        return pl.BlockSpec(w.shape, lambda b, _n=nd: (0,) * _n)

    in_specs = [
        bspec((B, N, 4)), bspec((B, A, 4)), bspec((B, A * N, 1)),
        bspec((B, G, 4)), bspec((B, G, 4)),
    ] + [wspec(w) for w in wleaves]

    return pl.pallas_call(
        body,
        grid=(B // PB,),
        in_specs=in_specs,
        out_specs=pl.BlockSpec((PB, A, N), lambda b: (b, 0, 0)),
        out_shape=jax.ShapeDtypeStruct((B, A, N), jnp.float32),
        compiler_params=pltpu.CompilerParams(
            dimension_semantics=("parallel",)),
    )(ghost_node_position, agent_world_pos, dis1,
      last_ghost_node_position, last_agent_world_pos, *wleaves)
